# Initial kernel scaffold; baseline (speedup 1.0000x reference)
import jax
import jax.numpy as jnp
from jax.experimental import pallas as pl


def kernel(x, edge_index, batch, assignment_index_2, iso_type_2, edge_index_2, batch_2, assignment_index_3, iso_type_3, edge_index_3, batch_3, W1r, W1n, b1, W2r, W2n, b2, W3r, W3n, b3, W4r, W4n, b4, W5r, W5n, b5, W6r, W6n, b6, W7r, W7n, b7, fc1_W, fc1_b, fc2_W, fc2_b, fc3_W, fc3_b):
    raise NotImplementedError("write your pallas kernel here")


# speedup vs baseline: 3.3173x; 3.3173x over previous
import functools

import jax
import jax.numpy as jnp
from jax import lax
from jax.experimental import pallas as pl
from jax.experimental.pallas import tpu as pltpu
from jax.experimental.pallas import tpu_sc as plsc

N_NODES = 10000
N_GRAPHS = 256
NC = 2
NS = 16
NW = NC * NS
_NBUF = 4


def _rup(x, m):
    return (x + m - 1) // m * m


@functools.lru_cache(None)
def _seg_sum_fn(e_pad, n_pad, w, t_rows):
    n_jw = e_pad // (128 * NW)
    assert e_pad % (128 * NW * _NBUF) == 0
    assert n_pad % (128 * NS) == 0
    rps = n_pad // NS
    nz = rps // 128
    f32 = jnp.float32

    mesh = plsc.VectorSubcoreMesh(core_axis_name="c", subcore_axis_name="s",
                                  num_cores=NC, num_subcores=NS)

    def body(table, src2d, dst2d, out0, out1, srcv, dstv, rows, zerov, acc, *sems):
        c = lax.axis_index("c")
        s = lax.axis_index("s")
        wid = c * NS + s

        def zrow(r, carry):
            for k in range(w // 16):
                zerov[r, pl.ds(k * 16, 16)] = jnp.zeros((16,), f32)
            return carry
        lax.fori_loop(0, 128, zrow, 0)

        row0 = s * rps
        for m in range(nz):
            pltpu.sync_copy(zerov, acc.at[pl.ds(row0 + m * 128, 128)])
        plsc.subcore_barrier()

        jbase = wid * n_jw
        pltpu.sync_copy(src2d.at[pl.ds(jbase, n_jw)], srcv)
        pltpu.sync_copy(dst2d.at[pl.ds(jbase, n_jw)], dstv)

        for b in range(_NBUF):
            pltpu.async_copy(table.at[srcv.at[b]], rows.at[b], sems[b])

        def step(jo, carry):
            for b in range(_NBUF):
                j = jo * _NBUF + b
                pltpu.make_async_copy(table.at[srcv.at[j]], rows.at[b], sems[b]).wait()
                pltpu.sync_copy(rows.at[b], acc.at[dstv.at[j]], add=True)

                @pl.when(j + _NBUF < n_jw)
                def _issue():
                    pltpu.async_copy(table.at[srcv.at[j + _NBUF]], rows.at[b], sems[b])
            return carry
        lax.fori_loop(0, n_jw // _NBUF, step, 0)

        plsc.subcore_barrier()

        @pl.when(c == 0)
        def _w0():
            for m in range(nz):
                pltpu.sync_copy(acc.at[pl.ds(row0 + m * 128, 128)],
                                out0.at[pl.ds(row0 + m * 128, 128)])

        @pl.when(c == 1)
        def _w1():
            for m in range(nz):
                pltpu.sync_copy(acc.at[pl.ds(row0 + m * 128, 128)],
                                out1.at[pl.ds(row0 + m * 128, 128)])

    return pl.kernel(
        body,
        out_type=(jax.ShapeDtypeStruct((n_pad, w), f32),
                  jax.ShapeDtypeStruct((n_pad, w), f32)),
        mesh=mesh,
        scratch_types=[
            pltpu.VMEM((n_jw, 128), jnp.int32),
            pltpu.VMEM((n_jw, 128), jnp.int32),
            pltpu.VMEM((_NBUF, 128, w), f32),
            pltpu.VMEM((128, w), f32),
            pltpu.VMEM_SHARED((n_pad, w), f32),
        ] + [pltpu.SemaphoreType.DMA] * _NBUF,
        compiler_params=pltpu.CompilerParams(use_tc_tiling_on_sc=False),
    )


def _seg_sum(table, src, dst, n_out):
    e = src.shape[0]
    e_pad = _rup(e, 128 * NW * _NBUF)
    n_pad = _rup(n_out + 1, 128 * NS)
    if e_pad != e:
        pad = e_pad - e
        src = jnp.concatenate([src, jnp.zeros((pad,), src.dtype)])
        dst = jnp.concatenate([dst, jnp.full((pad,), n_out, dst.dtype)])
    fn = _seg_sum_fn(e_pad, n_pad, int(table.shape[1]), int(table.shape[0]))
    return fn(table, src.reshape(-1, 128), dst.reshape(-1, 128))


def _elu(v):
    return jnp.where(v > 0, v, jnp.exp(v) - 1.0)


def _wspecs(kin, kout, bpg):
    return [
        pl.BlockSpec((1, kin, kout), lambda i: (i // bpg, 0, 0)),
        pl.BlockSpec((1, kin, kout), lambda i: (i // bpg, 0, 0)),
        pl.BlockSpec((1, 1, kout), lambda i: (i // bpg, 0, 0)),
    ]


def _dual_mm(x, wn, wr, b, br):
    n, kin = x.shape
    g, _, kout = wn.shape
    grid = n // br
    bpg = grid // g

    def body(x_ref, wn_ref, wr_ref, b_ref, y_ref, r_ref):
        xb = x_ref[...]
        y_ref[...] = jnp.dot(xb, wn_ref[0], preferred_element_type=jnp.float32)
        r_ref[...] = jnp.dot(xb, wr_ref[0], preferred_element_type=jnp.float32) + b_ref[0]

    return pl.pallas_call(
        body, grid=(grid,),
        in_specs=[pl.BlockSpec((br, kin), lambda i: (i, 0))] + _wspecs(kin, kout, bpg),
        out_specs=[pl.BlockSpec((br, kout), lambda i: (i, 0))] * 2,
        out_shape=[jax.ShapeDtypeStruct((n, kout), jnp.float32)] * 2,
    )(x, wn, wr, b)


def _combine_mm(p0, p1, r, wn, wr, b, br):
    n, kin = r.shape
    g, _, kout = wn.shape
    grid = n // br
    bpg = grid // g

    def body(p0_ref, p1_ref, r_ref, wn_ref, wr_ref, b_ref, y_ref, r2_ref):
        h = _elu(r_ref[...] + p0_ref[...] + p1_ref[...])
        y_ref[...] = jnp.dot(h, wn_ref[0], preferred_element_type=jnp.float32)
        r2_ref[...] = jnp.dot(h, wr_ref[0], preferred_element_type=jnp.float32) + b_ref[0]

    return pl.pallas_call(
        body, grid=(grid,),
        in_specs=[pl.BlockSpec((br, kin), lambda i: (i, 0))] * 3 + _wspecs(kin, kout, bpg),
        out_specs=[pl.BlockSpec((br, kout), lambda i: (i, 0))] * 2,
        out_shape=[jax.ShapeDtypeStruct((n, kout), jnp.float32)] * 2,
    )(p0, p1, r, wn, wr, b)


def _combine_aug(p0, p1, r, br):
    n, kin = r.shape

    def body(p0_ref, p1_ref, r_ref, o_ref):
        h = _elu(r_ref[...] + p0_ref[...] + p1_ref[...])
        o_ref[...] = jnp.concatenate(
            [h, jnp.ones((br, 1), jnp.float32), jnp.zeros((br, 15), jnp.float32)],
            axis=1)

    return pl.pallas_call(
        body, grid=(n // br,),
        in_specs=[pl.BlockSpec((br, kin), lambda i: (i, 0))] * 3,
        out_specs=pl.BlockSpec((br, 80), lambda i: (i, 0)),
        out_shape=jax.ShapeDtypeStruct((n, 80), jnp.float32),
    )(p0, p1, r)


def _pool_mm(q0, q1, iso, wn, wr, b, br):
    n, kiso = iso.shape
    g, kin, kout = wn.shape
    grid = n // br
    bpg = grid // g

    def body(q0_ref, q1_ref, iso_ref, wn_ref, wr_ref, b_ref, y_ref, r_ref):
        m = q0_ref[...] + q1_ref[...]
        mean = m[:, :64] / jnp.maximum(m[:, 64:65], 1.0)
        pf = jnp.concatenate([mean, iso_ref[...]], axis=1)
        y_ref[...] = jnp.dot(pf, wn_ref[0], preferred_element_type=jnp.float32)
        r_ref[...] = jnp.dot(pf, wr_ref[0], preferred_element_type=jnp.float32) + b_ref[0]

    return pl.pallas_call(
        body, grid=(grid,),
        in_specs=[pl.BlockSpec((br, 80), lambda i: (i, 0))] * 2
        + [pl.BlockSpec((br, kiso), lambda i: (i, 0))] + _wspecs(kin, kout, bpg),
        out_specs=[pl.BlockSpec((br, kout), lambda i: (i, 0))] * 2,
        out_shape=[jax.ShapeDtypeStruct((n, kout), jnp.float32)] * 2,
    )(q0, q1, iso, wn, wr, b)


def _head(parts, w1, b1, w2, b2, w3, b3):
    def body(p10, p11, p20, p21, p30, p31, w1_ref, b1_ref, w2_ref, b2_ref,
             w3_ref, b3_ref, o_ref):
        m1 = p10[...] + p11[...]
        m2 = p20[...] + p21[...]
        m3 = p30[...] + p31[...]
        x1 = m1[:, 0:64]
        x2 = m2[:, 0:64] / jnp.maximum(m2[:, 64:65], 1.0)
        x3 = m3[:, 0:64] / jnp.maximum(m3[:, 64:65], 1.0)
        xc = jnp.concatenate([x1, x2, x3], axis=1)
        h = _elu(jnp.dot(xc, w1_ref[...], preferred_element_type=jnp.float32) + b1_ref[...])
        h = _elu(jnp.dot(h, w2_ref[...], preferred_element_type=jnp.float32) + b2_ref[...])
        o_ref[...] = jnp.dot(h, w3_ref[...], preferred_element_type=jnp.float32) + b3_ref[...]

    return pl.pallas_call(
        body, grid=(1,),
        in_specs=[pl.BlockSpec((256, 80), lambda i: (0, 0))] * 6 + [
            pl.BlockSpec((192, 64), lambda i: (0, 0)),
            pl.BlockSpec((1, 64), lambda i: (0, 0)),
            pl.BlockSpec((64, 32), lambda i: (0, 0)),
            pl.BlockSpec((1, 32), lambda i: (0, 0)),
            pl.BlockSpec((32, 1), lambda i: (0, 0)),
            pl.BlockSpec((1, 1), lambda i: (0, 0)),
        ],
        out_specs=pl.BlockSpec((256, 1), lambda i: (0, 0)),
        out_shape=jax.ShapeDtypeStruct((256, 1), jnp.float32),
    )(*parts, w1, b1, w2, b2, w3, b3)


def kernel(x, edge_index, batch, assignment_index_2, iso_type_2, edge_index_2,
           batch_2, assignment_index_3, iso_type_3, edge_index_3, batch_3,
           W1r, W1n, b1, W2r, W2n, b2, W3r, W3n, b3, W4r, W4n, b4,
           W5r, W5n, b5, W6r, W6n, b6, W7r, W7n, b7,
           fc1_W, fc1_b, fc2_W, fc2_b, fc3_W, fc3_b):
    i32 = jnp.int32
    src1, dst1 = edge_index[0], edge_index[1]

    y, r = _dual_mm(x, W1n[None], W1r[None], b1[None, None], 1000)
    p0, p1 = _seg_sum(y, src1, dst1, N_NODES)
    y, r = _combine_mm(p0, p1, r, W2n[None], W2r[None], b2[None, None], 1000)
    p0, p1 = _seg_sum(y, src1, dst1, N_NODES)
    y, r = _combine_mm(p0, p1, r, W3n[None], W3r[None], b3[None, None], 1000)
    p0, p1 = _seg_sum(y, src1, dst1, N_NODES)
    haug = _combine_aug(p0, p1, r, 1000)

    src_r = jnp.arange(N_NODES, dtype=i32)

    def branch(ai, iso, eidx, bat, Wan, War, ba, Wbn, Wbr, bb):
        q0, q1 = _seg_sum(haug, ai[0], ai[1], N_NODES)
        y, r = _pool_mm(q0, q1, iso, Wan[None], War[None], ba[None, None], 1000)
        p0, p1 = _seg_sum(y, eidx[0], eidx[1], N_NODES)
        y, r = _combine_mm(p0, p1, r, Wbn[None], Wbr[None], bb[None, None], 1000)
        p0, p1 = _seg_sum(y, eidx[0], eidx[1], N_NODES)
        paug = _combine_aug(p0, p1, r, 1000)
        return _seg_sum(paug, src_r, bat.astype(i32), N_GRAPHS)

    rp2 = branch(assignment_index_2, iso_type_2, edge_index_2, batch_2,
                 W4n, W4r, b4, W5n, W5r, b5)
    rp3 = branch(assignment_index_3, iso_type_3, edge_index_3, batch_3,
                 W6n, W6r, b6, W7n, W7r, b7)
    rp1_ = _seg_sum(haug, src_r, batch.astype(i32), N_GRAPHS)

    out = _head([rp1_[0], rp1_[1], rp2[0], rp2[1], rp3[0], rp3[1]],
                fc1_W, fc1_b[None], fc2_W, fc2_b[None], fc3_W, fc3_b[None])
    return out.reshape(-1)


# speedup vs baseline: 3.6455x; 1.0989x over previous
import functools

import jax
import jax.numpy as jnp
from jax import lax
from jax.experimental import pallas as pl
from jax.experimental.pallas import tpu as pltpu
from jax.experimental.pallas import tpu_sc as plsc

N_NODES = 10000
N_GRAPHS = 256
NC = 1
NS = 16
NW = NC * NS
_NBUF = 4


def _rup(x, m):
    return (x + m - 1) // m * m


@functools.lru_cache(None)
def _seg_sum_fn(e_pad, n_pad, w, t_rows):
    n_jw = e_pad // (128 * NW)
    assert e_pad % (128 * NW * _NBUF) == 0
    assert n_pad % (128 * NS) == 0
    rps = n_pad // NS
    nz = rps // 128
    f32 = jnp.float32

    mesh = plsc.VectorSubcoreMesh(core_axis_name="c", subcore_axis_name="s",
                                  num_cores=NC, num_subcores=NS)

    def body(table, src2d, dst2d, out0, srcv, dstv, rows, zerov, acc, *sems):
        s = lax.axis_index("s")
        wid = s

        def zrow(r, carry):
            for k in range(w // 16):
                zerov[r, pl.ds(k * 16, 16)] = jnp.zeros((16,), f32)
            return carry
        lax.fori_loop(0, 128, zrow, 0)

        row0 = s * rps
        for m in range(nz):
            pltpu.sync_copy(zerov, acc.at[pl.ds(row0 + m * 128, 128)])
        plsc.subcore_barrier()

        jbase = wid * n_jw
        pltpu.sync_copy(src2d.at[pl.ds(jbase, n_jw)], srcv)
        pltpu.sync_copy(dst2d.at[pl.ds(jbase, n_jw)], dstv)

        for b in range(_NBUF):
            pltpu.async_copy(table.at[srcv.at[b]], rows.at[b], sems[b])

        def step(jo, carry):
            for b in range(_NBUF):
                j = jo * _NBUF + b
                pltpu.make_async_copy(table.at[srcv.at[j]], rows.at[b], sems[b]).wait()
                pltpu.sync_copy(rows.at[b], acc.at[dstv.at[j]], add=True)

                @pl.when(j + _NBUF < n_jw)
                def _issue():
                    pltpu.async_copy(table.at[srcv.at[j + _NBUF]], rows.at[b], sems[b])
            return carry
        lax.fori_loop(0, n_jw // _NBUF, step, 0)

        plsc.subcore_barrier()

        for m in range(nz):
            pltpu.sync_copy(acc.at[pl.ds(row0 + m * 128, 128)],
                            out0.at[pl.ds(row0 + m * 128, 128)])

    return pl.kernel(
        body,
        out_type=jax.ShapeDtypeStruct((n_pad, w), f32),
        mesh=mesh,
        scratch_types=[
            pltpu.VMEM((n_jw, 128), jnp.int32),
            pltpu.VMEM((n_jw, 128), jnp.int32),
            pltpu.VMEM((_NBUF, 128, w), f32),
            pltpu.VMEM((128, w), f32),
            pltpu.VMEM_SHARED((n_pad, w), f32),
        ] + [pltpu.SemaphoreType.DMA] * _NBUF,
        compiler_params=pltpu.CompilerParams(use_tc_tiling_on_sc=False),
    )


def _seg_sum(table, src, dst, n_out):
    e = src.shape[0]
    e_pad = _rup(e, 128 * NW * _NBUF)
    n_pad = _rup(n_out + 1, 128 * NS)
    if e_pad != e:
        pad = e_pad - e
        src = jnp.concatenate([src, jnp.zeros((pad,), src.dtype)])
        dst = jnp.concatenate([dst, jnp.full((pad,), n_out, dst.dtype)])
    fn = _seg_sum_fn(e_pad, n_pad, int(table.shape[1]), int(table.shape[0]))
    return fn(table, src.reshape(-1, 128), dst.reshape(-1, 128))


def _elu(v):
    return jnp.where(v > 0, v, jnp.exp(v) - 1.0)


def _wspecs(kin, kout, bpg):
    return [
        pl.BlockSpec((1, kin, kout), lambda i: (i // bpg, 0, 0)),
        pl.BlockSpec((1, kin, kout), lambda i: (i // bpg, 0, 0)),
        pl.BlockSpec((1, 1, kout), lambda i: (i // bpg, 0, 0)),
    ]


def _dual_mm(x, wn, wr, b, br):
    n, kin = x.shape
    g, _, kout = wn.shape
    grid = n // br
    bpg = grid // g

    def body(x_ref, wn_ref, wr_ref, b_ref, y_ref, r_ref):
        xb = x_ref[...]
        y_ref[...] = jnp.dot(xb, wn_ref[0], preferred_element_type=jnp.float32)
        r_ref[...] = jnp.dot(xb, wr_ref[0], preferred_element_type=jnp.float32) + b_ref[0]

    return pl.pallas_call(
        body, grid=(grid,),
        in_specs=[pl.BlockSpec((br, kin), lambda i: (i, 0))] + _wspecs(kin, kout, bpg),
        out_specs=[pl.BlockSpec((br, kout), lambda i: (i, 0))] * 2,
        out_shape=[jax.ShapeDtypeStruct((n, kout), jnp.float32)] * 2,
    )(x, wn, wr, b)


def _combine_mm(p0, r, wn, wr, b, br):
    n, kin = r.shape
    g, _, kout = wn.shape
    grid = n // br
    bpg = grid // g

    def body(p0_ref, r_ref, wn_ref, wr_ref, b_ref, y_ref, r2_ref):
        h = _elu(r_ref[...] + p0_ref[...])
        y_ref[...] = jnp.dot(h, wn_ref[0], preferred_element_type=jnp.float32)
        r2_ref[...] = jnp.dot(h, wr_ref[0], preferred_element_type=jnp.float32) + b_ref[0]

    return pl.pallas_call(
        body, grid=(grid,),
        in_specs=[pl.BlockSpec((br, kin), lambda i: (i, 0))] * 2 + _wspecs(kin, kout, bpg),
        out_specs=[pl.BlockSpec((br, kout), lambda i: (i, 0))] * 2,
        out_shape=[jax.ShapeDtypeStruct((n, kout), jnp.float32)] * 2,
    )(p0, r, wn, wr, b)


def _combine_aug(p0, r, br):
    n, kin = r.shape

    def body(p0_ref, r_ref, o_ref):
        h = _elu(r_ref[...] + p0_ref[...])
        o_ref[...] = jnp.concatenate(
            [h, jnp.ones((br, 1), jnp.float32), jnp.zeros((br, 15), jnp.float32)],
            axis=1)

    return pl.pallas_call(
        body, grid=(n // br,),
        in_specs=[pl.BlockSpec((br, kin), lambda i: (i, 0))] * 2,
        out_specs=pl.BlockSpec((br, 80), lambda i: (i, 0)),
        out_shape=jax.ShapeDtypeStruct((n, 80), jnp.float32),
    )(p0, r)


def _pool_mm(q0, iso, wn, wr, b, br):
    n, kiso = iso.shape
    g, kin, kout = wn.shape
    grid = n // br
    bpg = grid // g

    def body(q0_ref, iso_ref, wn_ref, wr_ref, b_ref, y_ref, r_ref):
        m = q0_ref[...]
        mean = m[:, :64] / jnp.maximum(m[:, 64:65], 1.0)
        pf = jnp.concatenate([mean, iso_ref[...]], axis=1)
        y_ref[...] = jnp.dot(pf, wn_ref[0], preferred_element_type=jnp.float32)
        r_ref[...] = jnp.dot(pf, wr_ref[0], preferred_element_type=jnp.float32) + b_ref[0]

    return pl.pallas_call(
        body, grid=(grid,),
        in_specs=[pl.BlockSpec((br, 80), lambda i: (i, 0))]
        + [pl.BlockSpec((br, kiso), lambda i: (i, 0))] + _wspecs(kin, kout, bpg),
        out_specs=[pl.BlockSpec((br, kout), lambda i: (i, 0))] * 2,
        out_shape=[jax.ShapeDtypeStruct((n, kout), jnp.float32)] * 2,
    )(q0, iso, wn, wr, b)


def _head(parts, w1, b1, w2, b2, w3, b3):
    def body(p10, p20, p30, w1_ref, b1_ref, w2_ref, b2_ref,
             w3_ref, b3_ref, o_ref):
        m1 = p10[...]
        m2 = p20[...]
        m3 = p30[...]
        x1 = m1[:, 0:64]
        x2 = m2[:, 0:64] / jnp.maximum(m2[:, 64:65], 1.0)
        x3 = m3[:, 0:64] / jnp.maximum(m3[:, 64:65], 1.0)
        xc = jnp.concatenate([x1, x2, x3], axis=1)
        h = _elu(jnp.dot(xc, w1_ref[...], preferred_element_type=jnp.float32) + b1_ref[...])
        h = _elu(jnp.dot(h, w2_ref[...], preferred_element_type=jnp.float32) + b2_ref[...])
        o_ref[...] = jnp.dot(h, w3_ref[...], preferred_element_type=jnp.float32) + b3_ref[...]

    return pl.pallas_call(
        body, grid=(1,),
        in_specs=[pl.BlockSpec((256, 80), lambda i: (0, 0))] * 3 + [
            pl.BlockSpec((192, 64), lambda i: (0, 0)),
            pl.BlockSpec((1, 64), lambda i: (0, 0)),
            pl.BlockSpec((64, 32), lambda i: (0, 0)),
            pl.BlockSpec((1, 32), lambda i: (0, 0)),
            pl.BlockSpec((32, 1), lambda i: (0, 0)),
            pl.BlockSpec((1, 1), lambda i: (0, 0)),
        ],
        out_specs=pl.BlockSpec((256, 1), lambda i: (0, 0)),
        out_shape=jax.ShapeDtypeStruct((256, 1), jnp.float32),
    )(*parts, w1, b1, w2, b2, w3, b3)


def kernel(x, edge_index, batch, assignment_index_2, iso_type_2, edge_index_2,
           batch_2, assignment_index_3, iso_type_3, edge_index_3, batch_3,
           W1r, W1n, b1, W2r, W2n, b2, W3r, W3n, b3, W4r, W4n, b4,
           W5r, W5n, b5, W6r, W6n, b6, W7r, W7n, b7,
           fc1_W, fc1_b, fc2_W, fc2_b, fc3_W, fc3_b):
    i32 = jnp.int32
    src1, dst1 = edge_index[0], edge_index[1]

    y, r = _dual_mm(x, W1n[None], W1r[None], b1[None, None], 1000)
    p = _seg_sum(y, src1, dst1, N_NODES)
    y, r = _combine_mm(p, r, W2n[None], W2r[None], b2[None, None], 1000)
    p = _seg_sum(y, src1, dst1, N_NODES)
    y, r = _combine_mm(p, r, W3n[None], W3r[None], b3[None, None], 1000)
    p = _seg_sum(y, src1, dst1, N_NODES)
    haug = _combine_aug(p, r, 1000)

    src_r = jnp.arange(N_NODES, dtype=i32)

    def branch(ai, iso, eidx, bat, Wan, War, ba, Wbn, Wbr, bb):
        q = _seg_sum(haug, ai[0], ai[1], N_NODES)
        y, r = _pool_mm(q, iso, Wan[None], War[None], ba[None, None], 1000)
        p = _seg_sum(y, eidx[0], eidx[1], N_NODES)
        y, r = _combine_mm(p, r, Wbn[None], Wbr[None], bb[None, None], 1000)
        p = _seg_sum(y, eidx[0], eidx[1], N_NODES)
        paug = _combine_aug(p, r, 1000)
        return _seg_sum(paug, src_r, bat.astype(i32), N_GRAPHS)

    rp2 = branch(assignment_index_2, iso_type_2, edge_index_2, batch_2,
                 W4n, W4r, b4, W5n, W5r, b5)
    rp3 = branch(assignment_index_3, iso_type_3, edge_index_3, batch_3,
                 W6n, W6r, b6, W7n, W7r, b7)
    rp1_ = _seg_sum(haug, src_r, batch.astype(i32), N_GRAPHS)

    out = _head([rp1_, rp2, rp3],
                fc1_W, fc1_b[None], fc2_W, fc2_b[None], fc3_W, fc3_b[None])
    return out.reshape(-1)
